# Initial kernel scaffold; baseline (speedup 1.0000x reference)
import jax
import jax.numpy as jnp
from jax.experimental import pallas as pl


def kernel(x, Wp, bp, Wo, bo, Ws, Wr, max_ops):
    raise NotImplementedError("write your pallas kernel here")


# speedup vs baseline: 2.0492x; 2.0492x over previous
import functools
import jax
import jax.numpy as jnp
from jax.experimental import pallas as pl
from jax.experimental.pallas import tpu as pltpu

B = 4096
D = 1024
E = 8
SYM = 128
HOPS = 4
TB = 256
NTB = B // TB


def _stage_a_body(x_ref, wp_ref, bp_ref, ws_ref, wrz_ref, wrs_ref,
                  z_ref, sym_ref, prog_ref, eff_ref):
    x = x_ref[...]
    z = jnp.dot(x, wp_ref[...], preferred_element_type=jnp.float32) + bp_ref[...]
    z_ref[...] = z
    sym = jnp.tanh(jnp.dot(z, ws_ref[...], preferred_element_type=jnp.float32))
    sym_ref[...] = sym
    msym = jnp.mean(sym.reshape(TB, E, SYM), axis=1)
    logits = (jnp.dot(z, wrz_ref[...], preferred_element_type=jnp.float32)
              + jnp.dot(msym, wrs_ref[...], preferred_element_type=jnp.float32))
    lg = logits.reshape(TB, HOPS, E + 1)
    mx = jnp.max(lg, axis=-1, keepdims=True)
    k_iota = jax.lax.broadcasted_iota(jnp.int32, (TB, HOPS, E + 1), 2)
    idx = jnp.min(jnp.where(lg >= mx, k_iota, E + 1), axis=-1).astype(jnp.int32)
    prog_ref[...] = idx
    active = jnp.ones((TB, 1), dtype=jnp.bool_)
    effs = []
    for h in range(HOPS):
        ph = idx[:, h:h + 1]
        ok = active & (ph != E)
        effs.append(jnp.where(ok, ph, E).astype(jnp.int32))
        active = ok
    eff_ref[...] = jnp.concatenate(effs, axis=1)


def _stage_a(x, Wp, bp2, Ws2, Wrz, Wrs):
    return pl.pallas_call(
        _stage_a_body,
        grid=(NTB,),
        in_specs=[
            pl.BlockSpec((TB, D), lambda i: (i, 0)),
            pl.BlockSpec((D, D), lambda i: (0, 0)),
            pl.BlockSpec((1, D), lambda i: (0, 0)),
            pl.BlockSpec((D, E * SYM), lambda i: (0, 0)),
            pl.BlockSpec((D, HOPS * (E + 1)), lambda i: (0, 0)),
            pl.BlockSpec((SYM, HOPS * (E + 1)), lambda i: (0, 0)),
        ],
        out_specs=[
            pl.BlockSpec((TB, D), lambda i: (i, 0)),
            pl.BlockSpec((TB, E * SYM), lambda i: (i, 0)),
            pl.BlockSpec((TB, HOPS), lambda i: (i, 0)),
            pl.BlockSpec((TB, HOPS), lambda i: (i, 0)),
        ],
        out_shape=[
            jax.ShapeDtypeStruct((B, D), jnp.float32),
            jax.ShapeDtypeStruct((B, E * SYM), jnp.float32),
            jax.ShapeDtypeStruct((B, HOPS), jnp.int32),
            jax.ShapeDtypeStruct((B, HOPS), jnp.int32),
        ],
    )(x, Wp, bp2, Ws2, Wrz, Wrs)


def _hop_body(h, in_ref, wo_ref, bo_ref, eff_ref, out_ref):
    z = in_ref[...]
    eff_h = eff_ref[:, h:h + 1]
    out = z
    for e in range(E):
        op = jnp.tanh(jnp.dot(z, wo_ref[e], preferred_element_type=jnp.float32)
                      + bo_ref[e:e + 1, :])
        out = jnp.where(eff_h == e, op, out)
    out_ref[...] = out


def _hop(out_prev, Wo, bo, eff, h):
    return pl.pallas_call(
        functools.partial(_hop_body, h),
        grid=(NTB,),
        in_specs=[
            pl.BlockSpec((TB, D), lambda i: (i, 0)),
            pl.BlockSpec((E, D, D), lambda i: (0, 0, 0)),
            pl.BlockSpec((E, D), lambda i: (0, 0)),
            pl.BlockSpec((TB, HOPS), lambda i: (i, 0)),
        ],
        out_specs=pl.BlockSpec((TB, D), lambda i: (i, 0)),
        out_shape=jax.ShapeDtypeStruct((B, D), jnp.float32),
    )(out_prev, Wo, bo, eff)


def kernel(x, Wp, bp, Wo, bo, Ws, Wr, max_ops):
    Ws2 = jnp.transpose(Ws, (1, 0, 2)).reshape(D, E * SYM)
    Wrz = jnp.transpose(Wr[:, :D, :], (1, 0, 2)).reshape(D, HOPS * (E + 1))
    Wrs = jnp.transpose(Wr[:, D:, :], (1, 0, 2)).reshape(SYM, HOPS * (E + 1))
    bp2 = bp.reshape(1, D)

    z, sym_flat, prog, eff = _stage_a(x, Wp, bp2, Ws2, Wrz, Wrs)

    out = z
    for h in range(HOPS):
        out = _hop(out, Wo, bo, eff, h)

    return out, prog, sym_flat.reshape(B, E, SYM)
